# baseline (speedup 1.0000x reference)
import jax
import jax.numpy as jnp
import numpy as np
from jax import lax
from jax.experimental import pallas as pl
from jax.experimental.pallas import tpu as pltpu
from jax.experimental.pallas import tpu_sc as plsc

_NC = 2
_NS = 16
_NW = _NC * _NS

_B = 4096
_L = 200
_D = 100
_DP = 128
_NBL = _DP // 32
_NCH = _DP // 16
_SPW = _B // _NW
_C1 = 104
_C2 = 96

_MLP_BLK = 512
_PAD_BLK = 4096

_PERM = np.zeros(_DP, np.int64)
for _k in range(_NBL):
    for _i in range(16):
        _PERM[32 * _k + _i] = 32 * _k + 2 * _i
        _PERM[32 * _k + 16 + _i] = 32 * _k + 2 * _i + 1


def _pool_body(x_hbm, table_hbm, out_hbm,
               idx_v, buf0, buf1, out_v, semA0, semB0, semA1, semB1):
    wid = lax.axis_index("c") * _NS + lax.axis_index("s")
    base = wid * _SPW
    pltpu.sync_copy(x_hbm.at[pl.ds(base, _SPW)], idx_v)
    bufs = (buf0, buf1)
    semsA = (semA0, semA1)
    semsB = (semB0, semB1)

    def fire(s, b):
        pltpu.async_copy(
            table_hbm.at[idx_v.at[s, pl.ds(0, _C1)]],
            bufs[b].at[pl.ds(0, _C1)], semsA[b])
        pltpu.async_copy(
            table_hbm.at[idx_v.at[s, pl.ds(_C1, _C2)]],
            bufs[b].at[pl.ds(_C1, _C2)], semsB[b])

    def wait_a(b):
        pltpu.make_async_copy(
            table_hbm.at[pl.ds(0, _C1)],
            bufs[b].at[pl.ds(0, _C1)], semsA[b]).wait()

    def wait_b(b):
        pltpu.make_async_copy(
            table_hbm.at[pl.ds(0, _C2)],
            bufs[b].at[pl.ds(_C1, _C2)], semsB[b]).wait()

    def acc_range(buf, row0, ngroups, accs):
        def group_body(g, accs):
            accs = list(accs)
            r0 = row0 + g * 4
            for rr in range(4):
                for c in range(_NBL):
                    v = buf[r0 + rr, pl.ds(32 * c, 32)]
                    w = plsc.bitcast(v, jnp.int32)
                    lo = plsc.bitcast(lax.shift_left(w, 16), jnp.float32)
                    hi = plsc.bitcast(
                        lax.bitwise_and(w, jnp.int32(-65536)), jnp.float32)
                    accs[2 * c] = accs[2 * c] + lo
                    accs[2 * c + 1] = accs[2 * c + 1] + hi
            return tuple(accs)
        return lax.fori_loop(0, ngroups, group_body, accs)

    fire(0, 0)

    def outer(i, carry):
        for b in range(2):
            s = 2 * i + b

            @pl.when(s + 1 < _SPW)
            def _():
                fire(s + 1, 1 - b)
            zero = jnp.zeros((16,), jnp.float32)
            wait_a(b)
            accs = acc_range(bufs[b], 0, _C1 // 4, (zero,) * _NCH)
            wait_b(b)
            accs = acc_range(bufs[b], _C1, _C2 // 4, accs)
            for c in range(_NCH):
                out_v[s, pl.ds(16 * c, 16)] = accs[c]
        return carry
    lax.fori_loop(0, _SPW // 2, outer, 0)
    pltpu.sync_copy(out_v, out_hbm.at[pl.ds(base, _SPW)])


def _pad_body(t_ref, o_ref):
    o_ref[:, :_D] = t_ref[...].astype(jnp.bfloat16)
    o_ref[:, _D:] = jnp.zeros((_PAD_BLK, _DP - _D), jnp.bfloat16)


def _mlp_body(acc_ref, w1_ref, b1_ref, w2_ref, b2_ref, out_ref):
    a = acc_ref[...]
    h = jnp.maximum(
        lax.dot(a, w1_ref[...], preferred_element_type=jnp.float32)
        + b1_ref[...], 0.0)
    logits = lax.dot(h, w2_ref[...], preferred_element_type=jnp.float32) \
        + b2_ref[...]
    m = jnp.max(logits, axis=1, keepdims=True)
    e = jnp.exp(logits - m)
    out_ref[...] = e / jnp.sum(e, axis=1, keepdims=True)


def kernel(x, table, W1, b1, W2, b2):
    assert x.shape == (_B, _L) and table.shape[1] == _D
    hid = W1.shape[1]
    out_d = W2.shape[1]
    vocab = table.shape[0]

    table_p = pl.pallas_call(
        _pad_body,
        grid=(pl.cdiv(vocab, _PAD_BLK),),
        in_specs=[pl.BlockSpec((_PAD_BLK, _D), lambda i: (i, 0))],
        out_specs=pl.BlockSpec((_PAD_BLK, _DP), lambda i: (i, 0)),
        out_shape=jax.ShapeDtypeStruct((vocab, _DP), jnp.bfloat16),
    )(table)

    mesh = plsc.VectorSubcoreMesh(
        core_axis_name="c", subcore_axis_name="s",
        num_cores=_NC, num_subcores=_NS)
    pool = pl.kernel(
        _pool_body,
        out_type=jax.ShapeDtypeStruct((_B, _DP), jnp.float32),
        mesh=mesh,
        scratch_types=[
            pltpu.VMEM((_SPW, _L), jnp.int32),
            pltpu.VMEM((_L, _DP), jnp.bfloat16),
            pltpu.VMEM((_L, _DP), jnp.bfloat16),
            pltpu.VMEM((_SPW, _DP), jnp.float32),
            pltpu.SemaphoreType.DMA,
            pltpu.SemaphoreType.DMA,
            pltpu.SemaphoreType.DMA,
            pltpu.SemaphoreType.DMA,
        ],
        compiler_params=pltpu.CompilerParams(
            use_tc_tiling_on_sc=False, needs_layout_passes=False),
    )
    acc = pool(x, table_p)

    w1x = jnp.concatenate(
        [W1, jnp.zeros((_DP - _D, hid), jnp.float32)], axis=0) * (1.0 / _L)
    w1m = w1x[_PERM]
    probs = pl.pallas_call(
        _mlp_body,
        grid=(_B // _MLP_BLK,),
        in_specs=[
            pl.BlockSpec((_MLP_BLK, _DP), lambda i: (i, 0)),
            pl.BlockSpec((_DP, hid), lambda i: (0, 0)),
            pl.BlockSpec((1, hid), lambda i: (0, 0)),
            pl.BlockSpec((hid, out_d), lambda i: (0, 0)),
            pl.BlockSpec((1, out_d), lambda i: (0, 0)),
        ],
        out_specs=pl.BlockSpec((_MLP_BLK, out_d), lambda i: (i, 0)),
        out_shape=jax.ShapeDtypeStruct((_B, out_d), jnp.float32),
    )(acc, w1m, b1.reshape(1, hid), W2, b2.reshape(1, out_d))
    return probs


# speedup vs baseline: 1.0234x; 1.0234x over previous
import jax
import jax.numpy as jnp
import numpy as np
from jax import lax
from jax.experimental import pallas as pl
from jax.experimental.pallas import tpu as pltpu
from jax.experimental.pallas import tpu_sc as plsc

_NC = 2
_NS = 16
_NW = _NC * _NS

_B = 4096
_L = 200
_D = 100
_DP = 128
_NBL = _DP // 32
_NCH = _DP // 16
_SPW = _B // _NW
_C1 = 104
_C2 = 96

_MLP_BLK = 512
_PAD_BLK = 4096

_PERM = np.zeros(_DP, np.int64)
for _k in range(_NBL):
    for _i in range(16):
        _PERM[32 * _k + _i] = 32 * _k + 2 * _i
        _PERM[32 * _k + 16 + _i] = 32 * _k + 2 * _i + 1


def _pool_body(x_hbm, table_hbm, out_hbm,
               idx_v, buf0, buf1, out_v, semA0, semB0, semA1, semB1):
    wid = lax.axis_index("c") * _NS + lax.axis_index("s")
    base = wid * _SPW
    pltpu.sync_copy(x_hbm.at[pl.ds(base, _SPW)], idx_v)
    bufs = (buf0, buf1)
    semsA = (semA0, semA1)
    semsB = (semB0, semB1)

    def fire(s, b):
        pltpu.async_copy(
            table_hbm.at[idx_v.at[s, pl.ds(0, _C1)]],
            bufs[b].at[pl.ds(0, _C1)], semsA[b])
        pltpu.async_copy(
            table_hbm.at[idx_v.at[s, pl.ds(_C1, _C2)]],
            bufs[b].at[pl.ds(_C1, _C2)], semsB[b])

    def wait_a(b):
        pltpu.make_async_copy(
            table_hbm.at[pl.ds(0, _C1)],
            bufs[b].at[pl.ds(0, _C1)], semsA[b]).wait()

    def wait_b(b):
        pltpu.make_async_copy(
            table_hbm.at[pl.ds(0, _C2)],
            bufs[b].at[pl.ds(_C1, _C2)], semsB[b]).wait()

    def acc_range(buf, row0, ngroups, accs):
        def group_body(g, accs):
            accs = list(accs)
            r0 = row0 + g * 8
            bz = jnp.zeros((32,), jnp.bfloat16)
            bacc = [bz] * _NBL
            for rr in range(8):
                for c in range(_NBL):
                    bacc[c] = bacc[c] + buf[r0 + rr, pl.ds(32 * c, 32)]
            for c in range(_NBL):
                w = plsc.bitcast(bacc[c], jnp.int32)
                lo = plsc.bitcast(lax.shift_left(w, 16), jnp.float32)
                hi = plsc.bitcast(
                    lax.bitwise_and(w, jnp.int32(-65536)), jnp.float32)
                accs[2 * c] = accs[2 * c] + lo
                accs[2 * c + 1] = accs[2 * c + 1] + hi
            return tuple(accs)
        return lax.fori_loop(0, ngroups, group_body, accs)

    fire(0, 0)

    def outer(i, carry):
        for b in range(2):
            s = 2 * i + b

            @pl.when(s + 1 < _SPW)
            def _():
                fire(s + 1, 1 - b)
            zero = jnp.zeros((16,), jnp.float32)
            wait_a(b)
            accs = acc_range(bufs[b], 0, _C1 // 8, (zero,) * _NCH)
            wait_b(b)
            accs = acc_range(bufs[b], _C1, _C2 // 8, accs)
            for c in range(_NCH):
                out_v[s, pl.ds(16 * c, 16)] = accs[c]
        return carry
    lax.fori_loop(0, _SPW // 2, outer, 0)
    pltpu.sync_copy(out_v, out_hbm.at[pl.ds(base, _SPW)])


def _pad_body(t_ref, o_ref):
    o_ref[:, :_D] = t_ref[...].astype(jnp.bfloat16)
    o_ref[:, _D:] = jnp.zeros((_PAD_BLK, _DP - _D), jnp.bfloat16)


def _mlp_body(acc_ref, w1_ref, b1_ref, w2_ref, b2_ref, out_ref):
    a = acc_ref[...]
    h = jnp.maximum(
        lax.dot(a, w1_ref[...], preferred_element_type=jnp.float32)
        + b1_ref[...], 0.0)
    logits = lax.dot(h, w2_ref[...], preferred_element_type=jnp.float32) \
        + b2_ref[...]
    m = jnp.max(logits, axis=1, keepdims=True)
    e = jnp.exp(logits - m)
    out_ref[...] = e / jnp.sum(e, axis=1, keepdims=True)


def kernel(x, table, W1, b1, W2, b2):
    assert x.shape == (_B, _L) and table.shape[1] == _D
    hid = W1.shape[1]
    out_d = W2.shape[1]
    vocab = table.shape[0]

    table_p = pl.pallas_call(
        _pad_body,
        grid=(pl.cdiv(vocab, _PAD_BLK),),
        in_specs=[pl.BlockSpec((_PAD_BLK, _D), lambda i: (i, 0))],
        out_specs=pl.BlockSpec((_PAD_BLK, _DP), lambda i: (i, 0)),
        out_shape=jax.ShapeDtypeStruct((vocab, _DP), jnp.bfloat16),
    )(table)

    mesh = plsc.VectorSubcoreMesh(
        core_axis_name="c", subcore_axis_name="s",
        num_cores=_NC, num_subcores=_NS)
    pool = pl.kernel(
        _pool_body,
        out_type=jax.ShapeDtypeStruct((_B, _DP), jnp.float32),
        mesh=mesh,
        scratch_types=[
            pltpu.VMEM((_SPW, _L), jnp.int32),
            pltpu.VMEM((_L, _DP), jnp.bfloat16),
            pltpu.VMEM((_L, _DP), jnp.bfloat16),
            pltpu.VMEM((_SPW, _DP), jnp.float32),
            pltpu.SemaphoreType.DMA,
            pltpu.SemaphoreType.DMA,
            pltpu.SemaphoreType.DMA,
            pltpu.SemaphoreType.DMA,
        ],
        compiler_params=pltpu.CompilerParams(
            use_tc_tiling_on_sc=False, needs_layout_passes=False),
    )
    acc = pool(x, table_p)

    w1x = jnp.concatenate(
        [W1, jnp.zeros((_DP - _D, hid), jnp.float32)], axis=0) * (1.0 / _L)
    w1m = w1x[_PERM]
    probs = pl.pallas_call(
        _mlp_body,
        grid=(_B // _MLP_BLK,),
        in_specs=[
            pl.BlockSpec((_MLP_BLK, _DP), lambda i: (i, 0)),
            pl.BlockSpec((_DP, hid), lambda i: (0, 0)),
            pl.BlockSpec((1, hid), lambda i: (0, 0)),
            pl.BlockSpec((hid, out_d), lambda i: (0, 0)),
            pl.BlockSpec((1, out_d), lambda i: (0, 0)),
        ],
        out_specs=pl.BlockSpec((_MLP_BLK, out_d), lambda i: (i, 0)),
        out_shape=jax.ShapeDtypeStruct((_B, out_d), jnp.float32),
    )(acc, w1m, b1.reshape(1, hid), W2, b2.reshape(1, out_d))
    return probs
